# Initial kernel scaffold; baseline (speedup 1.0000x reference)
import jax
import jax.numpy as jnp
from jax.experimental import pallas as pl


def kernel(xc, yc, xt, pos, edge_index, enc_w0, enc_b0, enc_w1, enc_b1, enc_w2, enc_b2, gcn_w, gcn_b, dec_w0, dec_b0, dec_w1, dec_b1, dec_w2, dec_b2):
    raise NotImplementedError("write your pallas kernel here")


# speedup vs baseline: 9.4729x; 9.4729x over previous
import jax
import jax.numpy as jnp
from jax import lax
from jax.experimental import pallas as pl
from jax.experimental.pallas import tpu as pltpu
from jax.experimental.pallas import tpu_sc as plsc

N = 10000
NP = 10240
E = 320000
B = 2
NC = 1024
NT = 1024
DH = 128
STEPS = 3
NBLK = NP // 1024
RB = 2048
CK = 80
NCHUNK = E // CK
ROWS_PER_SUB = NP // 16

_f32 = jnp.float32


def _enc_body(xcp_ref, cin_ref, posp_ref, w0_ref, b0_ref, w1_ref, b1_ref,
              w2_ref, b2_ref, lat_ref):
    xcp = xcp_ref[0]
    cin = cin_ref[0]
    h = jnp.maximum(jnp.dot(cin, w0_ref[...],
                            preferred_element_type=_f32) + b0_ref[...], 0.0)
    h = jnp.maximum(jnp.dot(h, w1_ref[...],
                            preferred_element_type=_f32) + b1_ref[...], 0.0)
    emb = jnp.dot(h, w2_ref[...], preferred_element_type=_f32) + b2_ref[...]
    xn2 = jnp.sum(xcp * xcp, axis=1, keepdims=True)

    def p1(j, carry):
        m, s = carry
        pb = posp_ref[pl.ds(j * 1024, 1024), :]
        pn2 = jnp.sum(pb * pb, axis=1)
        g = lax.dot_general(xcp, pb, (((1,), (1,)), ((), ())),
                            preferred_element_type=_f32)
        sc = 2.0 * g - xn2 - pn2[None, :]
        mb = jnp.max(sc, axis=1, keepdims=True)
        mn = jnp.maximum(m, mb)
        s = s * jnp.exp(m - mn) + jnp.sum(jnp.exp(sc - mn), axis=1,
                                          keepdims=True)
        return mn, s

    m0 = jnp.full((NC, 1), -1e30, _f32)
    s0 = jnp.zeros((NC, 1), _f32)
    m, s = lax.fori_loop(0, NBLK, p1, (m0, s0))
    mls = (m + jnp.log(s))[:, 0]
    xn2r = xn2[:, 0]

    def p2(j, _):
        pb = posp_ref[pl.ds(j * 1024, 1024), :]
        pn2 = jnp.sum(pb * pb, axis=1, keepdims=True)
        g2 = lax.dot_general(pb, xcp, (((1,), (1,)), ((), ())),
                             preferred_element_type=_f32)
        eb = jnp.exp(2.0 * g2 - pn2 - xn2r[None, :] - mls[None, :])
        lat_ref[0, pl.ds(j * 1024, 1024), :] = jnp.dot(
            eb, emb, preferred_element_type=_f32)
        return 0

    lax.fori_loop(0, NBLK, p2, 0)


def _enc_call(xcp, cin, posp, w0p, b0, w1, b1, w2, b2):
    return pl.pallas_call(
        _enc_body,
        grid=(B,),
        in_specs=[
            pl.BlockSpec((1, NC, 8), lambda b: (b, 0, 0)),
            pl.BlockSpec((1, NC, 8), lambda b: (b, 0, 0)),
            pl.BlockSpec((NP, 8), lambda b: (0, 0)),
            pl.BlockSpec((8, DH), lambda b: (0, 0)),
            pl.BlockSpec((DH,), lambda b: (0,)),
            pl.BlockSpec((DH, DH), lambda b: (0, 0)),
            pl.BlockSpec((DH,), lambda b: (0,)),
            pl.BlockSpec((DH, DH), lambda b: (0, 0)),
            pl.BlockSpec((DH,), lambda b: (0,)),
        ],
        out_specs=pl.BlockSpec((1, NP, DH), lambda b: (b, 0, 0)),
        out_shape=jax.ShapeDtypeStruct((B, NP, DH), _f32),
    )(xcp, cin, posp, w0p, b0, w1, b1, w2, b2)


def _dinv_from_hist(hist_ref):
    deg = hist_ref[0, :, 0] + hist_ref[1, :, 0] + 1.0
    return lax.rsqrt(deg)[:, None]


def _mm_body(x_ref, pos2_ref, hist_ref, wh_ref, wp_ref, xs_ref):
    xw = (jnp.dot(x_ref[...], wh_ref[...], preferred_element_type=_f32)
          + jnp.dot(pos2_ref[...], wp_ref[...], preferred_element_type=_f32))
    xs_ref[...] = _dinv_from_hist(hist_ref) * xw


def _comb_body(acc_ref, xsp_ref, pos2_ref, hist_ref, gb_ref, wh_ref, wp_ref,
               xs_ref):
    dinv = _dinv_from_hist(hist_ref)
    x = dinv * (acc_ref[...] + xsp_ref[...]) + gb_ref[...]
    xw = (jnp.dot(x, wh_ref[...], preferred_element_type=_f32)
          + jnp.dot(pos2_ref[...], wp_ref[...], preferred_element_type=_f32))
    xs_ref[...] = dinv * xw


_G = (B * NP) // RB


def _row_specs():
    return [
        pl.BlockSpec((RB, DH), lambda p: (p, 0)),
        pl.BlockSpec((RB, 8), lambda p: (p, 0)),
        pl.BlockSpec((2, RB, 16), lambda p: (0, p % (NP // RB), 0)),
    ]


def _mm_call(x, pos2, histp, wh, wp):
    return pl.pallas_call(
        _mm_body,
        grid=(_G,),
        in_specs=[
            _row_specs()[0],
            _row_specs()[1],
            _row_specs()[2],
            pl.BlockSpec((DH, DH), lambda p: (0, 0)),
            pl.BlockSpec((8, DH), lambda p: (0, 0)),
        ],
        out_specs=pl.BlockSpec((RB, DH), lambda p: (p, 0)),
        out_shape=jax.ShapeDtypeStruct((B * NP, DH), _f32),
    )(x, pos2, histp, wh, wp)


def _comb_call(acc, xsp, pos2, histp, gb, wh, wp):
    return pl.pallas_call(
        _comb_body,
        grid=(_G,),
        in_specs=[
            pl.BlockSpec((RB, DH), lambda p: (p, 0)),
            _row_specs()[0],
            _row_specs()[1],
            _row_specs()[2],
            pl.BlockSpec((DH,), lambda p: (0,)),
            pl.BlockSpec((DH, DH), lambda p: (0, 0)),
            pl.BlockSpec((8, DH), lambda p: (0, 0)),
        ],
        out_specs=pl.BlockSpec((RB, DH), lambda p: (p, 0)),
        out_shape=jax.ShapeDtypeStruct((B * NP, DH), _f32),
    )(acc, xsp, pos2, histp, gb, wh, wp)


def _dec_body(xtp_ref, posp_ref, acc3_ref, xs3_ref, hist_ref, gb_ref,
              d0z_ref, d0x_ref, db0_ref, d1_ref, db1_ref, d2p_ref, db2p_ref,
              out_ref):
    xt = xtp_ref[0]
    xt2 = jnp.sum(xt * xt, axis=1, keepdims=True)

    def blk(j, carry):
        m, l, acc = carry
        pb = posp_ref[pl.ds(j * 1024, 1024), :]
        pn2 = jnp.sum(pb * pb, axis=1)
        deg = (hist_ref[0, pl.ds(j * 1024, 1024), 0]
               + hist_ref[1, pl.ds(j * 1024, 1024), 0] + 1.0)
        dinv = lax.rsqrt(deg)[:, None]
        xb = dinv * (acc3_ref[0, pl.ds(j * 1024, 1024), :]
                     + xs3_ref[0, pl.ds(j * 1024, 1024), :]) + gb_ref[...]
        sc = 2.0 * lax.dot_general(xt, pb, (((1,), (1,)), ((), ())),
                                   preferred_element_type=_f32) \
            - xt2 - pn2[None, :]
        mb = jnp.max(sc, axis=1, keepdims=True)
        mn = jnp.maximum(m, mb)
        alpha = jnp.exp(m - mn)
        p = jnp.exp(sc - mn)
        acc = acc * alpha + jnp.dot(p, xb, preferred_element_type=_f32)
        l = l * alpha + jnp.sum(p, axis=1, keepdims=True)
        return mn, l, acc

    m0 = jnp.full((NT, 1), -1e30, _f32)
    l0 = jnp.zeros((NT, 1), _f32)
    a0 = jnp.zeros((NT, DH), _f32)
    m, l, acc = lax.fori_loop(0, NBLK, blk, (m0, l0, a0))
    z = acc / l
    h = jnp.maximum(jnp.dot(z, d0z_ref[...], preferred_element_type=_f32)
                    + jnp.dot(xt, d0x_ref[...], preferred_element_type=_f32)
                    + db0_ref[...], 0.0)
    h = jnp.maximum(jnp.dot(h, d1_ref[...], preferred_element_type=_f32)
                    + db1_ref[...], 0.0)
    out_ref[0] = jnp.dot(h, d2p_ref[...],
                         preferred_element_type=_f32) + db2p_ref[...]


def _dec_call(xtp, posp, acc3, xs3, histp, gb, d0z, d0x, db0, d1, db1,
              d2p, db2p):
    return pl.pallas_call(
        _dec_body,
        grid=(B,),
        in_specs=[
            pl.BlockSpec((1, NT, 8), lambda b: (b, 0, 0)),
            pl.BlockSpec((NP, 8), lambda b: (0, 0)),
            pl.BlockSpec((1, NP, DH), lambda b: (b, 0, 0)),
            pl.BlockSpec((1, NP, DH), lambda b: (b, 0, 0)),
            pl.BlockSpec((2, NP, 16), lambda b: (0, 0, 0)),
            pl.BlockSpec((DH,), lambda b: (0,)),
            pl.BlockSpec((DH, DH), lambda b: (0, 0)),
            pl.BlockSpec((8, DH), lambda b: (0, 0)),
            pl.BlockSpec((DH,), lambda b: (0,)),
            pl.BlockSpec((DH, DH), lambda b: (0, 0)),
            pl.BlockSpec((DH,), lambda b: (0,)),
            pl.BlockSpec((DH, DH), lambda b: (0, 0)),
            pl.BlockSpec((DH,), lambda b: (0,)),
        ],
        out_specs=pl.BlockSpec((1, NT, DH), lambda b: (b, 0, 0)),
        out_shape=jax.ShapeDtypeStruct((B, NT, DH), _f32),
    )(xtp, posp, acc3, xs3, histp, gb, d0z, d0x, db0, d1, db1, d2p, db2p)


def _sc_mesh():
    return plsc.VectorSubcoreMesh(core_axis_name="c", subcore_axis_name="s",
                                  num_cores=2, num_subcores=16)


def _deg_body(dst2, hist, acc16, idxb, ones16, zbuf):
    c = lax.axis_index("c")
    s = lax.axis_index("s")

    def fill(i, _):
        ones16[i] = jnp.full((16,), 1.0, _f32)
        zbuf[i] = jnp.zeros((16,), _f32)
        return 0
    lax.fori_loop(0, CK, fill, 0)

    def zero(i, _):
        pltpu.sync_copy(zbuf, acc16.at[pl.ds(s * ROWS_PER_SUB + i * CK, CK)])
        return 0
    lax.fori_loop(0, ROWS_PER_SUB // CK, zero, 0)
    plsc.subcore_barrier()

    w = c * 16 + s

    def body(k, _):
        pltpu.sync_copy(dst2.at[w * (NCHUNK // 32) + k], idxb)
        pltpu.sync_copy(ones16, acc16.at[idxb], add=True)
        return 0
    lax.fori_loop(0, NCHUNK // 32, body, 0)
    plsc.subcore_barrier()
    pltpu.sync_copy(acc16.at[pl.ds(s * ROWS_PER_SUB, ROWS_PER_SUB)],
                    hist.at[c, pl.ds(s * ROWS_PER_SUB, ROWS_PER_SUB)])


def _deg_call(dst2):
    return pl.kernel(
        _deg_body,
        out_type=jax.ShapeDtypeStruct((2, NP, 16), _f32),
        mesh=_sc_mesh(),
        scratch_types=[
            pltpu.VMEM_SHARED((NP, 16), _f32),
            pltpu.VMEM((CK,), jnp.int32),
            pltpu.VMEM((CK, 16), _f32),
            pltpu.VMEM((CK, 16), _f32),
        ],
    )(dst2)


def _edge_body(srcoff3, dst2, xs, out, acc, sidx, didx, rows, zbuf, sem):
    c = lax.axis_index("c")
    s = lax.axis_index("s")

    def fz(i, _):
        zbuf[i // 8, pl.ds((i % 8) * 16, 16)] = jnp.zeros((16,), _f32)
        return 0
    lax.fori_loop(0, CK * 8, fz, 0)

    def zero(i, _):
        pltpu.sync_copy(zbuf, acc.at[pl.ds(s * ROWS_PER_SUB + i * CK, CK)])
        return 0
    lax.fori_loop(0, ROWS_PER_SUB // CK, zero, 0)
    plsc.subcore_barrier()

    nck = NCHUNK // 16

    def body(k, _):
        row = s * nck + k
        pltpu.sync_copy(srcoff3.at[c, row], sidx)
        pltpu.async_copy(xs.at[sidx], rows, sem).wait()
        pltpu.sync_copy(dst2.at[row], didx)
        pltpu.sync_copy(rows, acc.at[didx], add=True)
        return 0
    lax.fori_loop(0, nck, body, 0)
    plsc.subcore_barrier()
    pltpu.sync_copy(acc.at[pl.ds(s * ROWS_PER_SUB, ROWS_PER_SUB)],
                    out.at[c, pl.ds(s * ROWS_PER_SUB, ROWS_PER_SUB)])


def _edge_call(srcoff3, dst2, xs):
    return pl.kernel(
        _edge_body,
        out_type=jax.ShapeDtypeStruct((2, NP, DH), _f32),
        mesh=_sc_mesh(),
        scratch_types=[
            pltpu.VMEM_SHARED((NP, DH), _f32),
            pltpu.VMEM((CK,), jnp.int32),
            pltpu.VMEM((CK,), jnp.int32),
            pltpu.VMEM((CK, DH), _f32),
            pltpu.VMEM((CK, DH), _f32),
            pltpu.SemaphoreType.DMA,
        ],
    )(srcoff3, dst2, xs)


def kernel(xc, yc, xt, pos, edge_index, enc_w0, enc_b0, enc_w1, enc_b1,
           enc_w2, enc_b2, gcn_w, gcn_b, dec_w0, dec_b0, dec_w1, dec_b1,
           dec_w2, dec_b2):
    posp = (jnp.zeros((NP, 8), _f32).at[:N, :3].set(pos)
            .at[N:, :3].set(1e4))
    xcp = jnp.zeros((B, NC, 8), _f32).at[:, :, :3].set(xc)
    cin = xcp.at[:, :, 3:7].set(yc)
    xtp = jnp.zeros((B, NT, 8), _f32).at[:, :, :3].set(xt)
    w0p = jnp.zeros((8, DH), _f32).at[:7].set(enc_w0)
    wh = gcn_w[:DH]
    wp = jnp.zeros((8, DH), _f32).at[:3].set(gcn_w[DH:])
    d0z = dec_w0[:DH]
    d0x = jnp.zeros((8, DH), _f32).at[:3].set(dec_w0[DH:])
    d2p = jnp.zeros((DH, DH), _f32).at[:, :4].set(dec_w2)
    db2p = jnp.zeros((DH,), _f32).at[:4].set(dec_b2)
    src = edge_index[0]
    dst = edge_index[1]
    srcoff3 = jnp.stack([src, src + NP]).reshape(2, NCHUNK, CK)
    dst2 = dst.reshape(NCHUNK, CK)
    pos2 = jnp.tile(posp, (B, 1))

    histp = _deg_call(dst2)
    lat0 = _enc_call(xcp, cin, posp, w0p, enc_b0, enc_w1, enc_b1,
                     enc_w2, enc_b2)
    xs = _mm_call(lat0.reshape(B * NP, DH), pos2, histp, wh, wp)
    acc = None
    for step in range(STEPS):
        acc = _edge_call(srcoff3, dst2, xs)
        if step < STEPS - 1:
            xs = _comb_call(acc.reshape(B * NP, DH), xs, pos2, histp,
                            gcn_b, wh, wp)
    out = _dec_call(xtp, posp, acc, xs.reshape(B, NP, DH), histp, gcn_b,
                    d0z, d0x, dec_b0, dec_w1, dec_b1, d2p, db2p)
    return out[:, :, :4]


# speedup vs baseline: 9.7317x; 1.0273x over previous
import jax
import jax.numpy as jnp
from jax import lax
from jax.experimental import pallas as pl
from jax.experimental.pallas import tpu as pltpu
from jax.experimental.pallas import tpu_sc as plsc

N = 10000
NP = 10240
E = 320000
B = 2
NC = 1024
NT = 1024
DH = 128
STEPS = 3
NBLK = NP // 1024
RB = 2048
CK = 80
NCHUNK = E // CK
ROWS_PER_SUB = NP // 16
EK = 128
NCKS = 160
EPAD = 16 * NCKS * EK
NBUF = 2

_f32 = jnp.float32


def _enc_body(xcp_ref, cin_ref, posp_ref, w0_ref, b0_ref, w1_ref, b1_ref,
              w2_ref, b2_ref, lat_ref):
    xcp = xcp_ref[0]
    cin = cin_ref[0]
    h = jnp.maximum(jnp.dot(cin, w0_ref[...],
                            preferred_element_type=_f32) + b0_ref[...], 0.0)
    h = jnp.maximum(jnp.dot(h, w1_ref[...],
                            preferred_element_type=_f32) + b1_ref[...], 0.0)
    emb = jnp.dot(h, w2_ref[...], preferred_element_type=_f32) + b2_ref[...]
    xn2 = jnp.sum(xcp * xcp, axis=1, keepdims=True)

    def p1(j, carry):
        m, s = carry
        pb = posp_ref[pl.ds(j * 1024, 1024), :]
        pn2 = jnp.sum(pb * pb, axis=1)
        g = lax.dot_general(xcp, pb, (((1,), (1,)), ((), ())),
                            preferred_element_type=_f32)
        sc = 2.0 * g - xn2 - pn2[None, :]
        mb = jnp.max(sc, axis=1, keepdims=True)
        mn = jnp.maximum(m, mb)
        s = s * jnp.exp(m - mn) + jnp.sum(jnp.exp(sc - mn), axis=1,
                                          keepdims=True)
        return mn, s

    m0 = jnp.full((NC, 1), -1e30, _f32)
    s0 = jnp.zeros((NC, 1), _f32)
    m, s = lax.fori_loop(0, NBLK, p1, (m0, s0))
    mls = (m + jnp.log(s))[:, 0]
    xn2r = xn2[:, 0]

    def p2(j, _):
        pb = posp_ref[pl.ds(j * 1024, 1024), :]
        pn2 = jnp.sum(pb * pb, axis=1, keepdims=True)
        g2 = lax.dot_general(pb, xcp, (((1,), (1,)), ((), ())),
                             preferred_element_type=_f32)
        eb = jnp.exp(2.0 * g2 - pn2 - xn2r[None, :] - mls[None, :])
        lat_ref[0, pl.ds(j * 1024, 1024), :] = jnp.dot(
            eb, emb, preferred_element_type=_f32)
        return 0

    lax.fori_loop(0, NBLK, p2, 0)


def _enc_call(xcp, cin, posp, w0p, b0, w1, b1, w2, b2):
    return pl.pallas_call(
        _enc_body,
        grid=(B,),
        in_specs=[
            pl.BlockSpec((1, NC, 8), lambda b: (b, 0, 0)),
            pl.BlockSpec((1, NC, 8), lambda b: (b, 0, 0)),
            pl.BlockSpec((NP, 8), lambda b: (0, 0)),
            pl.BlockSpec((8, DH), lambda b: (0, 0)),
            pl.BlockSpec((DH,), lambda b: (0,)),
            pl.BlockSpec((DH, DH), lambda b: (0, 0)),
            pl.BlockSpec((DH,), lambda b: (0,)),
            pl.BlockSpec((DH, DH), lambda b: (0, 0)),
            pl.BlockSpec((DH,), lambda b: (0,)),
        ],
        out_specs=pl.BlockSpec((1, NP, DH), lambda b: (b, 0, 0)),
        out_shape=jax.ShapeDtypeStruct((B, NP, DH), _f32),
    )(xcp, cin, posp, w0p, b0, w1, b1, w2, b2)


def _dinv_from_hist(hist_ref):
    deg = hist_ref[0, :, 0] + hist_ref[1, :, 0] + 1.0
    return lax.rsqrt(deg)[:, None]


def _mm_body(x_ref, pos2_ref, hist_ref, wh_ref, wp_ref, xs_ref):
    xw = (jnp.dot(x_ref[...], wh_ref[...], preferred_element_type=_f32)
          + jnp.dot(pos2_ref[...], wp_ref[...], preferred_element_type=_f32))
    xs_ref[...] = _dinv_from_hist(hist_ref) * xw


def _comb_body(acc_ref, xsp_ref, pos2_ref, hist_ref, gb_ref, wh_ref, wp_ref,
               xs_ref):
    dinv = _dinv_from_hist(hist_ref)
    x = dinv * (acc_ref[...] + xsp_ref[...]) + gb_ref[...]
    xw = (jnp.dot(x, wh_ref[...], preferred_element_type=_f32)
          + jnp.dot(pos2_ref[...], wp_ref[...], preferred_element_type=_f32))
    xs_ref[...] = dinv * xw


_G = (B * NP) // RB


def _row_specs():
    return [
        pl.BlockSpec((RB, DH), lambda p: (p, 0)),
        pl.BlockSpec((RB, 8), lambda p: (p, 0)),
        pl.BlockSpec((2, RB, 16), lambda p: (0, p % (NP // RB), 0)),
    ]


def _mm_call(x, pos2, histp, wh, wp):
    return pl.pallas_call(
        _mm_body,
        grid=(_G,),
        in_specs=[
            _row_specs()[0],
            _row_specs()[1],
            _row_specs()[2],
            pl.BlockSpec((DH, DH), lambda p: (0, 0)),
            pl.BlockSpec((8, DH), lambda p: (0, 0)),
        ],
        out_specs=pl.BlockSpec((RB, DH), lambda p: (p, 0)),
        out_shape=jax.ShapeDtypeStruct((B * NP, DH), _f32),
    )(x, pos2, histp, wh, wp)


def _comb_call(acc, xsp, pos2, histp, gb, wh, wp):
    return pl.pallas_call(
        _comb_body,
        grid=(_G,),
        in_specs=[
            pl.BlockSpec((RB, DH), lambda p: (p, 0)),
            _row_specs()[0],
            _row_specs()[1],
            _row_specs()[2],
            pl.BlockSpec((DH,), lambda p: (0,)),
            pl.BlockSpec((DH, DH), lambda p: (0, 0)),
            pl.BlockSpec((8, DH), lambda p: (0, 0)),
        ],
        out_specs=pl.BlockSpec((RB, DH), lambda p: (p, 0)),
        out_shape=jax.ShapeDtypeStruct((B * NP, DH), _f32),
    )(acc, xsp, pos2, histp, gb, wh, wp)


def _dec_body(xtp_ref, posp_ref, acc3_ref, xs3_ref, hist_ref, gb_ref,
              d0z_ref, d0x_ref, db0_ref, d1_ref, db1_ref, d2p_ref, db2p_ref,
              out_ref):
    xt = xtp_ref[0]
    xt2 = jnp.sum(xt * xt, axis=1, keepdims=True)

    def blk(j, carry):
        m, l, acc = carry
        pb = posp_ref[pl.ds(j * 1024, 1024), :]
        pn2 = jnp.sum(pb * pb, axis=1)
        deg = (hist_ref[0, pl.ds(j * 1024, 1024), 0]
               + hist_ref[1, pl.ds(j * 1024, 1024), 0] + 1.0)
        dinv = lax.rsqrt(deg)[:, None]
        xb = dinv * (acc3_ref[0, pl.ds(j * 1024, 1024), :]
                     + xs3_ref[0, pl.ds(j * 1024, 1024), :]) + gb_ref[...]
        sc = 2.0 * lax.dot_general(xt, pb, (((1,), (1,)), ((), ())),
                                   preferred_element_type=_f32) \
            - xt2 - pn2[None, :]
        mb = jnp.max(sc, axis=1, keepdims=True)
        mn = jnp.maximum(m, mb)
        alpha = jnp.exp(m - mn)
        p = jnp.exp(sc - mn)
        acc = acc * alpha + jnp.dot(p, xb, preferred_element_type=_f32)
        l = l * alpha + jnp.sum(p, axis=1, keepdims=True)
        return mn, l, acc

    m0 = jnp.full((NT, 1), -1e30, _f32)
    l0 = jnp.zeros((NT, 1), _f32)
    a0 = jnp.zeros((NT, DH), _f32)
    m, l, acc = lax.fori_loop(0, NBLK, blk, (m0, l0, a0))
    z = acc / l
    h = jnp.maximum(jnp.dot(z, d0z_ref[...], preferred_element_type=_f32)
                    + jnp.dot(xt, d0x_ref[...], preferred_element_type=_f32)
                    + db0_ref[...], 0.0)
    h = jnp.maximum(jnp.dot(h, d1_ref[...], preferred_element_type=_f32)
                    + db1_ref[...], 0.0)
    out_ref[0] = jnp.dot(h, d2p_ref[...],
                         preferred_element_type=_f32) + db2p_ref[...]


def _dec_call(xtp, posp, acc3, xs3, histp, gb, d0z, d0x, db0, d1, db1,
              d2p, db2p):
    return pl.pallas_call(
        _dec_body,
        grid=(B,),
        in_specs=[
            pl.BlockSpec((1, NT, 8), lambda b: (b, 0, 0)),
            pl.BlockSpec((NP, 8), lambda b: (0, 0)),
            pl.BlockSpec((1, NP, DH), lambda b: (b, 0, 0)),
            pl.BlockSpec((1, NP, DH), lambda b: (b, 0, 0)),
            pl.BlockSpec((2, NP, 16), lambda b: (0, 0, 0)),
            pl.BlockSpec((DH,), lambda b: (0,)),
            pl.BlockSpec((DH, DH), lambda b: (0, 0)),
            pl.BlockSpec((8, DH), lambda b: (0, 0)),
            pl.BlockSpec((DH,), lambda b: (0,)),
            pl.BlockSpec((DH, DH), lambda b: (0, 0)),
            pl.BlockSpec((DH,), lambda b: (0,)),
            pl.BlockSpec((DH, DH), lambda b: (0, 0)),
            pl.BlockSpec((DH,), lambda b: (0,)),
        ],
        out_specs=pl.BlockSpec((1, NT, DH), lambda b: (b, 0, 0)),
        out_shape=jax.ShapeDtypeStruct((B, NT, DH), _f32),
    )(xtp, posp, acc3, xs3, histp, gb, d0z, d0x, db0, d1, db1, d2p, db2p)


def _sc_mesh():
    return plsc.VectorSubcoreMesh(core_axis_name="c", subcore_axis_name="s",
                                  num_cores=2, num_subcores=16)


def _deg_body(dst2, hist, acc16, idxb, ones16, zbuf):
    c = lax.axis_index("c")
    s = lax.axis_index("s")

    def fill(i, _):
        ones16[i] = jnp.full((16,), 1.0, _f32)
        zbuf[i] = jnp.zeros((16,), _f32)
        return 0
    lax.fori_loop(0, CK, fill, 0)

    def zero(i, _):
        pltpu.sync_copy(zbuf, acc16.at[pl.ds(s * ROWS_PER_SUB + i * CK, CK)])
        return 0
    lax.fori_loop(0, ROWS_PER_SUB // CK, zero, 0)
    plsc.subcore_barrier()

    w = c * 16 + s

    def body(k, _):
        pltpu.sync_copy(dst2.at[w * (NCHUNK // 32) + k], idxb)
        pltpu.sync_copy(ones16, acc16.at[idxb], add=True)
        return 0
    lax.fori_loop(0, NCHUNK // 32, body, 0)
    plsc.subcore_barrier()
    pltpu.sync_copy(acc16.at[pl.ds(s * ROWS_PER_SUB, ROWS_PER_SUB)],
                    hist.at[c, pl.ds(s * ROWS_PER_SUB, ROWS_PER_SUB)])


def _deg_call(dst2):
    return pl.kernel(
        _deg_body,
        out_type=jax.ShapeDtypeStruct((2, NP, 16), _f32),
        mesh=_sc_mesh(),
        scratch_types=[
            pltpu.VMEM_SHARED((NP, 16), _f32),
            pltpu.VMEM((CK,), jnp.int32),
            pltpu.VMEM((CK, 16), _f32),
            pltpu.VMEM((CK, 16), _f32),
        ],
    )(dst2)


def _edge_body(edges5, xs, out, acc, sd0, sd1, rows0, rows1, sem0, sem1):
    c = lax.axis_index("c")
    s = lax.axis_index("s")
    rows = (rows0, rows1)
    sds = (sd0, sd1)
    sems = (sem0, sem1)

    def fz(i, _):
        rows0[i // 8, pl.ds((i % 8) * 16, 16)] = jnp.zeros((16,), _f32)
        return 0
    lax.fori_loop(0, EK * 8, fz, 0)

    def zero(i, _):
        pltpu.sync_copy(rows0, acc.at[pl.ds(s * ROWS_PER_SUB + i * EK, EK)])
        return 0
    lax.fori_loop(0, ROWS_PER_SUB // EK, zero, 0)
    plsc.subcore_barrier()

    def fetch(k, j):
        pltpu.sync_copy(edges5.at[c, s, k], sds[j])
        pltpu.async_copy(xs.at[sds[j].at[0]], rows[j], sems[j])

    def drain(j):
        pltpu.make_async_copy(xs.at[pl.ds(0, EK)], rows[j], sems[j]).wait()

    for j in range(NBUF):
        fetch(j, j)

    def body(i, _):
        for j in range(NBUF):
            k = i * NBUF + j
            drain(j)
            pltpu.sync_copy(rows[j], acc.at[sds[j].at[1]], add=True)

            @pl.when(k + NBUF < NCKS)
            def _():
                fetch(k + NBUF, j)
        return 0
    lax.fori_loop(0, NCKS // NBUF, body, 0)
    plsc.subcore_barrier()
    pltpu.sync_copy(acc.at[pl.ds(s * ROWS_PER_SUB, ROWS_PER_SUB)],
                    out.at[c, pl.ds(s * ROWS_PER_SUB, ROWS_PER_SUB)])


def _edge_call(edges5, xs):
    return pl.kernel(
        _edge_body,
        out_type=jax.ShapeDtypeStruct((2, NP, DH), _f32),
        mesh=_sc_mesh(),
        scratch_types=[
            pltpu.VMEM_SHARED((NP, DH), _f32),
            pltpu.VMEM((2, EK), jnp.int32),
            pltpu.VMEM((2, EK), jnp.int32),
            pltpu.VMEM((EK, DH), _f32),
            pltpu.VMEM((EK, DH), _f32),
            pltpu.SemaphoreType.DMA,
            pltpu.SemaphoreType.DMA,
        ],
    )(edges5, xs)


def kernel(xc, yc, xt, pos, edge_index, enc_w0, enc_b0, enc_w1, enc_b1,
           enc_w2, enc_b2, gcn_w, gcn_b, dec_w0, dec_b0, dec_w1, dec_b1,
           dec_w2, dec_b2):
    posp = (jnp.zeros((NP, 8), _f32).at[:N, :3].set(pos)
            .at[N:, :3].set(1e4))
    xcp = jnp.zeros((B, NC, 8), _f32).at[:, :, :3].set(xc)
    cin = xcp.at[:, :, 3:7].set(yc)
    xtp = jnp.zeros((B, NT, 8), _f32).at[:, :, :3].set(xt)
    w0p = jnp.zeros((8, DH), _f32).at[:7].set(enc_w0)
    wh = gcn_w[:DH]
    wp = jnp.zeros((8, DH), _f32).at[:3].set(gcn_w[DH:])
    d0z = dec_w0[:DH]
    d0x = jnp.zeros((8, DH), _f32).at[:3].set(dec_w0[DH:])
    d2p = jnp.zeros((DH, DH), _f32).at[:, :4].set(dec_w2)
    db2p = jnp.zeros((DH,), _f32).at[:4].set(dec_b2)
    src = edge_index[0]
    dst = edge_index[1]
    srcp = jnp.concatenate([src, jnp.zeros((EPAD - E,), jnp.int32)])
    dstp = jnp.concatenate([dst, jnp.full((EPAD - E,), NP - 1, jnp.int32)])
    edges5 = jnp.stack([
        jnp.stack([srcp.reshape(16, NCKS, EK),
                   dstp.reshape(16, NCKS, EK)], axis=2),
        jnp.stack([(srcp + NP).reshape(16, NCKS, EK),
                   dstp.reshape(16, NCKS, EK)], axis=2),
    ])
    dst2 = dst.reshape(NCHUNK, CK)
    pos2 = jnp.tile(posp, (B, 1))

    histp = _deg_call(dst2)
    lat0 = _enc_call(xcp, cin, posp, w0p, enc_b0, enc_w1, enc_b1,
                     enc_w2, enc_b2)
    xs = _mm_call(lat0.reshape(B * NP, DH), pos2, histp, wh, wp)
    acc = None
    for step in range(STEPS):
        acc = _edge_call(edges5, xs)
        if step < STEPS - 1:
            xs = _comb_call(acc.reshape(B * NP, DH), xs, pos2, histp,
                            gcn_b, wh, wp)
    out = _dec_call(xtp, posp, acc, xs.reshape(B, NP, DH), histp, gcn_b,
                    d0z, d0x, dec_b0, dec_w1, dec_b1, d2p, db2p)
    return out[:, :, :4]
